# baseline (speedup 1.0000x reference)
import functools

import jax
import jax.numpy as jnp
from jax import lax
from jax.experimental import pallas as pl
from jax.experimental.pallas import tpu as pltpu
from jax.experimental.pallas import tpu_sc as plsc

N = 10000
E = 320000
D = 128
H = 128
C = 16
R = 8
B = 4

NC = 2
NS = 16
NW = NC * NS
KB1 = 80
NITER1 = E // (NW * KB1)
KB2 = 128
NITER2 = -(-E // (NW * KB2))
EPAD = NW * NITER2 * KB2
N2 = N + 8
RPT = 624
RPT_LAST = N - (NS - 1) * RPT
RPT_LAST2 = N2 - (NS - 1) * RPT
SC_HALF = (NITER1 + 1) // 2
SC_STAGES = ((0, SC_HALF), (SC_HALF, NITER1 - SC_HALF))


def _striped_rows(sid, copy, last=RPT_LAST):
    @pl.when(sid < NS - 1)
    def _():
        copy(sid * RPT, RPT)

    @pl.when(sid == NS - 1)
    def _():
        copy((NS - 1) * RPT, last)


def _l1_body(x_ref, bases_ref, coeff_ref, wself_ref, b1_ref, et_ref, src_ref,
             table_ref, self_ref, o1_ref, o2_ref):
    xb = x_ref[...]
    for r in range(R):
        w = coeff_ref[r, 0] * bases_ref[0]
        for b in range(1, B):
            w = w + coeff_ref[r, b] * bases_ref[b]
        table_ref[r] = jnp.dot(xb, w, preferred_element_type=jnp.float32)
    self_ref[...] = (
        jnp.dot(xb, wself_ref[...], preferred_element_type=jnp.float32)
        + b1_ref[...]
    )

    @pl.when(pl.program_id(0) == 0)
    def _():
        o1_ref[...] = et_ref[...] * N + src_ref[...]
        o2_ref[...] = src_ref[...] * R + et_ref[...]


def _layer1_tables(x, bases1, coeff1, wself1, b1r, et2d, src2d):
    bn = 1000
    nb = N // bn
    nrows = et2d.shape[0]
    return pl.pallas_call(
        _l1_body,
        grid=(nb,),
        in_specs=[
            pl.BlockSpec((bn, D), lambda i: (i, 0)),
            pl.BlockSpec((B, D, H), lambda i: (0, 0, 0)),
            pl.BlockSpec((R, B), lambda i: (0, 0)),
            pl.BlockSpec((D, H), lambda i: (0, 0)),
            pl.BlockSpec((1, H), lambda i: (0, 0)),
            pl.BlockSpec((nrows, 128), lambda i: (0, 0)),
            pl.BlockSpec((nrows, 128), lambda i: (0, 0)),
        ],
        out_specs=[
            pl.BlockSpec((R, bn, H), lambda i: (0, i, 0)),
            pl.BlockSpec((bn, H), lambda i: (i, 0)),
            pl.BlockSpec((nrows, 128), lambda i: (0, 0)),
            pl.BlockSpec((nrows, 128), lambda i: (0, 0)),
        ],
        out_shape=[
            jax.ShapeDtypeStruct((R, N, H), jnp.float32),
            jax.ShapeDtypeStruct((N, H), jnp.float32),
            jax.ShapeDtypeStruct((nrows, 128), jnp.int32),
            jax.ShapeDtypeStruct((nrows, 128), jnp.int32),
        ],
    )(x, bases1, coeff1, wself1, b1r, et2d, src2d)


def _sc_l1_body(table_ref, gidx_ref, dst_ref, zagg_ref, zdeg_ref, ones_ref,
                agg_out, deg_out,
                gidx_all, dst_all, rows_v, ones_v, agg_sh, deg_sh,
                sem_a, sem_b, sem_oa, sem_ob):
    cid = lax.axis_index("c")
    sid = lax.axis_index("s")
    wid = sid * NC + cid
    sems = (sem_a, sem_b)
    osems = (sem_oa, sem_ob)

    def _zero(off, cnt):
        off = pl.multiple_of(off, 8)
        pltpu.sync_copy(zagg_ref.at[pl.ds(off, cnt)], agg_sh.at[pl.ds(off, cnt)])
        pltpu.sync_copy(zdeg_ref.at[pl.ds(off, cnt)], deg_sh.at[pl.ds(off, cnt)])

    _striped_rows(sid, _zero)
    pltpu.sync_copy(ones_ref, ones_v)
    plsc.subcore_barrier()

    for si, (start, sl) in enumerate(SC_STAGES):
        pltpu.sync_copy(gidx_ref.at[wid, pl.ds(start, sl)],
                        gidx_all.at[pl.ds(0, sl)])
        pltpu.sync_copy(dst_ref.at[wid, pl.ds(start, sl)],
                        dst_all.at[pl.ds(0, sl)])
        for b in range(2):
            pltpu.async_copy(table_ref.at[gidx_all.at[b]], rows_v.at[b], sems[b])

        def body(j, carry):
            for b in range(2):
                g = j * 2 + b

                @pl.when(g < sl)
                def _():
                    pltpu.make_async_copy(
                        table_ref.at[gidx_all.at[g]], rows_v.at[b],
                        sems[b]).wait()

                    @pl.when(jnp.logical_or(jnp.bool_(si > 0), g >= 2))
                    def _():
                        pltpu.make_async_copy(
                            ones_v, deg_sh.at[dst_all.at[g]], osems[b]).wait()

                    pltpu.async_copy(ones_v, deg_sh.at[dst_all.at[g]],
                                     osems[b], add=True)
                    pltpu.sync_copy(rows_v.at[b], agg_sh.at[dst_all.at[g]],
                                    add=True)

                @pl.when(g + 2 < sl)
                def _():
                    pltpu.async_copy(
                        table_ref.at[gidx_all.at[g + 2]], rows_v.at[b], sems[b])

            return carry

        lax.fori_loop(0, (sl + 1) // 2, body, 0)

    for b in range(2):
        pltpu.make_async_copy(ones_v, deg_sh.at[dst_all.at[b]], osems[b]).wait()
    plsc.subcore_barrier()

    def _out(off, cnt):
        off = pl.multiple_of(off, 8)
        pltpu.sync_copy(agg_sh.at[pl.ds(off, cnt)],
                        agg_out.at[cid, pl.ds(off, cnt)])
        pltpu.sync_copy(deg_sh.at[pl.ds(off, cnt)],
                        deg_out.at[cid, pl.ds(off, cnt)])

    _striped_rows(sid, _out)


def _sc_layer1(table1, gidx3, dst3, zagg, zdeg, ones):
    mesh = plsc.VectorSubcoreMesh(core_axis_name="c", subcore_axis_name="s")
    return pl.kernel(
        _sc_l1_body,
        mesh=mesh,
        out_type=[
            jax.ShapeDtypeStruct((NC, N, H), jnp.float32),
            jax.ShapeDtypeStruct((NC, N, 16), jnp.float32),
        ],
        scratch_types=[
            pltpu.VMEM((SC_HALF, KB1), jnp.int32),
            pltpu.VMEM((SC_HALF, KB1), jnp.int32),
            pltpu.VMEM((2, KB1, H), jnp.float32),
            pltpu.VMEM((KB1, 16), jnp.float32),
            pltpu.VMEM_SHARED((N, H), jnp.float32),
            pltpu.VMEM_SHARED((N, 16), jnp.float32),
            pltpu.SemaphoreType.DMA,
            pltpu.SemaphoreType.DMA,
            pltpu.SemaphoreType.DMA,
            pltpu.SemaphoreType.DMA,
        ],
        compiler_params=pltpu.CompilerParams(use_tc_tiling_on_sc=False),
    )(table1, gidx3, dst3, zagg, zdeg, ones)


def _l3_body(p1_ref, dp_ref, self1_ref, bases_ref, coeff_ref, wself_ref, b2_ref,
             table_ref, self2_ref):
    agg = p1_ref[0] + p1_ref[1]
    deg = dp_ref[0] + dp_ref[1]
    degc = jnp.max(deg, axis=1, keepdims=True)
    norm = 1.0 / jnp.maximum(degc, 1.0)
    h = jnp.maximum(agg * norm + self1_ref[...], 0.0)
    cols = []
    for r in range(R):
        w = coeff_ref[r, 0] * bases_ref[0]
        for b in range(1, B):
            w = w + coeff_ref[r, b] * bases_ref[b]
        cols.append(jnp.dot(h, w, preferred_element_type=jnp.float32))
    table_ref[...] = jnp.concatenate(cols, axis=1)
    self2_ref[...] = (
        jnp.dot(h, wself_ref[...], preferred_element_type=jnp.float32)
        + b2_ref[...]
    )


def _layer2_tables(p1, dp, self1, bases2, coeff2, wself2, b2r):
    bn = 1000
    nb = N // bn
    return pl.pallas_call(
        _l3_body,
        grid=(nb,),
        in_specs=[
            pl.BlockSpec((NC, bn, H), lambda i: (0, i, 0)),
            pl.BlockSpec((NC, bn, 16), lambda i: (0, i, 0)),
            pl.BlockSpec((bn, H), lambda i: (i, 0)),
            pl.BlockSpec((B, H, C), lambda i: (0, 0, 0)),
            pl.BlockSpec((R, B), lambda i: (0, 0)),
            pl.BlockSpec((H, C), lambda i: (0, 0)),
            pl.BlockSpec((1, C), lambda i: (0, 0)),
        ],
        out_specs=[
            pl.BlockSpec((bn, R * C), lambda i: (i, 0)),
            pl.BlockSpec((bn, C), lambda i: (i, 0)),
        ],
        out_shape=[
            jax.ShapeDtypeStruct((N, R * C), jnp.float32),
            jax.ShapeDtypeStruct((N, C), jnp.float32),
        ],
    )(p1, dp, self1, bases2, coeff2, wself2, b2r)


def _sc_l2_body(table_ref, gidx_ref, dst_ref, zagg_ref,
                agg_out,
                gidx_all, dst_all, rows_v, agg_sh, sem_a, sem_b):
    cid = lax.axis_index("c")
    sid = lax.axis_index("s")
    wid = sid * NC + cid
    sems = (sem_a, sem_b)

    pltpu.sync_copy(gidx_ref.at[wid], gidx_all)
    pltpu.sync_copy(dst_ref.at[wid], dst_all)

    def _zero(off, cnt):
        off = pl.multiple_of(off, 8)
        pltpu.sync_copy(zagg_ref.at[pl.ds(off, cnt)], agg_sh.at[pl.ds(off, cnt)])

    _striped_rows(sid, _zero, last=RPT_LAST2)
    plsc.subcore_barrier()

    for b in range(2):
        pltpu.async_copy(table_ref.at[gidx_all.at[b]], rows_v.at[b], sems[b])

    def body(j, carry):
        for b in range(2):
            g = j * 2 + b

            @pl.when(g < NITER2)
            def _():
                pltpu.make_async_copy(
                    table_ref.at[gidx_all.at[g]], rows_v.at[b], sems[b]).wait()
                pltpu.sync_copy(rows_v.at[b], agg_sh.at[dst_all.at[g]], add=True)

            @pl.when(g + 2 < NITER2)
            def _():
                pltpu.async_copy(
                    table_ref.at[gidx_all.at[g + 2]], rows_v.at[b], sems[b])

        return carry

    lax.fori_loop(0, (NITER2 + 1) // 2, body, 0)
    plsc.subcore_barrier()

    def _out(off, cnt):
        off = pl.multiple_of(off, 8)
        pltpu.sync_copy(agg_sh.at[pl.ds(off, cnt)],
                        agg_out.at[cid, pl.ds(off, cnt)])

    _striped_rows(sid, _out)


def _sc_layer2(table2, gidx3, dst3, zdeg):
    mesh = plsc.VectorSubcoreMesh(core_axis_name="c", subcore_axis_name="s")
    return pl.kernel(
        _sc_l2_body,
        mesh=mesh,
        out_type=jax.ShapeDtypeStruct((NC, N, C), jnp.float32),
        scratch_types=[
            pltpu.VMEM((NITER2, KB2), jnp.int32),
            pltpu.VMEM((NITER2, KB2), jnp.int32),
            pltpu.VMEM((2, KB2, C), jnp.float32),
            pltpu.VMEM_SHARED((N2, C), jnp.float32),
            pltpu.SemaphoreType.DMA,
            pltpu.SemaphoreType.DMA,
        ],
        compiler_params=pltpu.CompilerParams(use_tc_tiling_on_sc=False),
    )(table2, gidx3, dst3, zdeg)


def _l5_body(p2_ref, dp_ref, self2_ref, o_ref):
    agg = p2_ref[0] + p2_ref[1]
    deg = dp_ref[0] + dp_ref[1]
    norm = 1.0 / jnp.maximum(deg, 1.0)
    logits = agg * norm + self2_ref[...]
    m = jnp.max(logits, axis=1, keepdims=True)
    e = jnp.exp(logits - m)
    o_ref[...] = e / jnp.sum(e, axis=1, keepdims=True)


def _final(p2, dp, self2):
    bn = 1000
    nb = N // bn
    return pl.pallas_call(
        _l5_body,
        grid=(nb,),
        in_specs=[
            pl.BlockSpec((NC, bn, C), lambda i: (0, i, 0)),
            pl.BlockSpec((NC, bn, 16), lambda i: (0, i, 0)),
            pl.BlockSpec((bn, C), lambda i: (i, 0)),
        ],
        out_specs=pl.BlockSpec((bn, C), lambda i: (i, 0)),
        out_shape=jax.ShapeDtypeStruct((N, C), jnp.float32),
    )(p2, dp, self2)


@jax.jit
def kernel(x, edge_index, edge_type, bases1, coeff1, wself1, b1,
           bases2, coeff2, wself2, b2):
    src2d = edge_index[0].reshape(E // 128, 128)
    et2d = edge_type.reshape(E // 128, 128)
    table1, self1, gidx1, gidx2 = _layer1_tables(
        x, bases1, coeff1, wself1, b1.reshape(1, H), et2d, src2d)
    gidx1_3 = gidx1.reshape(NW, NITER1, KB1)
    dst3 = edge_index[1].reshape(NW, NITER1, KB1)
    gpad = jnp.zeros((EPAD - E,), jnp.int32)
    dpad = jnp.full((EPAD - E,), N, jnp.int32)
    gidx2_3 = jnp.concatenate([gidx2.reshape(E), gpad]).reshape(NW, NITER2, KB2)
    dst3b = jnp.concatenate([edge_index[1], dpad]).reshape(NW, NITER2, KB2)

    zagg = jnp.zeros((N, H), jnp.float32)
    zdeg = jnp.zeros((N, 16), jnp.float32)
    zdeg2 = jnp.zeros((N2, 16), jnp.float32)
    ones = jnp.ones((KB1, 16), jnp.float32)

    p1, dp = _sc_layer1(table1.reshape(R * N, H), gidx1_3, dst3, zagg, zdeg, ones)
    table2p, self2 = _layer2_tables(
        p1, dp, self1, bases2, coeff2, wself2, b2.reshape(1, C))
    p2 = _sc_layer2(table2p.reshape(N * R, C), gidx2_3, dst3b, zdeg2)
    return _final(p2, dp, self2)


# speedup vs baseline: 1.0185x; 1.0185x over previous
import functools

import jax
import jax.numpy as jnp
from jax import lax
from jax.experimental import pallas as pl
from jax.experimental.pallas import tpu as pltpu
from jax.experimental.pallas import tpu_sc as plsc

N = 10000
E = 320000
D = 128
H = 128
C = 16
R = 8
B = 4

NC = 2
NS = 16
NW = NC * NS
KB1 = 80
NITER1 = E // (NW * KB1)
KB2 = 128
NITER2 = -(-E // (NW * KB2))
EPAD = NW * NITER2 * KB2
N2 = N + 8
RPT = 624
RPT_LAST = N - (NS - 1) * RPT
RPT_LAST2 = N2 - (NS - 1) * RPT
SC_HALF = (NITER1 + 1) // 2
SC_STAGES = ((0, SC_HALF), (SC_HALF, NITER1 - SC_HALF))


def _striped_rows(sid, copy, last=RPT_LAST):
    @pl.when(sid < NS - 1)
    def _():
        copy(sid * RPT, RPT)

    @pl.when(sid == NS - 1)
    def _():
        copy((NS - 1) * RPT, last)


def _gidx_body(et_ref, src_ref, o1_ref, o2_ref):
    o1_ref[...] = et_ref[...] * N + src_ref[...]
    o2_ref[...] = src_ref[...] * R + et_ref[...]


def _gidx(et2d, src2d):
    nrows = et2d.shape[0]
    return pl.pallas_call(
        _gidx_body,
        grid=(1,),
        in_specs=[pl.BlockSpec((nrows, 128), lambda i: (0, 0))] * 2,
        out_specs=[pl.BlockSpec((nrows, 128), lambda i: (0, 0))] * 2,
        out_shape=[jax.ShapeDtypeStruct((nrows, 128), jnp.int32)] * 2,
    )(et2d, src2d)


def _l1_body(x_ref, bases_ref, coeff_ref, wself_ref, b1_ref, table_ref, self_ref):
    xb = x_ref[...]
    for r in range(R):
        w = coeff_ref[r, 0] * bases_ref[0]
        for b in range(1, B):
            w = w + coeff_ref[r, b] * bases_ref[b]
        table_ref[r] = jnp.dot(xb, w, preferred_element_type=jnp.float32)
    self_ref[...] = (
        jnp.dot(xb, wself_ref[...], preferred_element_type=jnp.float32)
        + b1_ref[...]
    )


def _layer1_tables(x, bases1, coeff1, wself1, b1r):
    bn = 1000
    nb = N // bn
    return pl.pallas_call(
        _l1_body,
        grid=(nb,),
        in_specs=[
            pl.BlockSpec((bn, D), lambda i: (i, 0)),
            pl.BlockSpec((B, D, H), lambda i: (0, 0, 0)),
            pl.BlockSpec((R, B), lambda i: (0, 0)),
            pl.BlockSpec((D, H), lambda i: (0, 0)),
            pl.BlockSpec((1, H), lambda i: (0, 0)),
        ],
        out_specs=[
            pl.BlockSpec((R, bn, H), lambda i: (0, i, 0)),
            pl.BlockSpec((bn, H), lambda i: (i, 0)),
        ],
        out_shape=[
            jax.ShapeDtypeStruct((R, N, H), jnp.float32),
            jax.ShapeDtypeStruct((N, H), jnp.float32),
        ],
    )(x, bases1, coeff1, wself1, b1r)


def _sc_l1_body(table_ref, gidx_ref, dst_ref, zagg_ref, zdeg_ref, ones_ref,
                agg_out, deg_out,
                gidx_all, dst_all, rows_v, ones_v, agg_sh, deg_sh,
                sem_a, sem_b, sem_oa, sem_ob):
    cid = lax.axis_index("c")
    sid = lax.axis_index("s")
    wid = sid * NC + cid
    sems = (sem_a, sem_b)
    osems = (sem_oa, sem_ob)

    def _zero(off, cnt):
        off = pl.multiple_of(off, 8)
        pltpu.sync_copy(zagg_ref.at[pl.ds(off, cnt)], agg_sh.at[pl.ds(off, cnt)])
        pltpu.sync_copy(zdeg_ref.at[pl.ds(off, cnt)], deg_sh.at[pl.ds(off, cnt)])

    _striped_rows(sid, _zero)
    pltpu.sync_copy(ones_ref, ones_v)
    plsc.subcore_barrier()

    for si, (start, sl) in enumerate(SC_STAGES):
        pltpu.sync_copy(gidx_ref.at[wid, pl.ds(start, sl)],
                        gidx_all.at[pl.ds(0, sl)])
        pltpu.sync_copy(dst_ref.at[wid, pl.ds(start, sl)],
                        dst_all.at[pl.ds(0, sl)])
        for b in range(2):
            pltpu.async_copy(table_ref.at[gidx_all.at[b]], rows_v.at[b], sems[b])

        def body(j, carry):
            for b in range(2):
                g = j * 2 + b

                @pl.when(g < sl)
                def _():
                    pltpu.make_async_copy(
                        table_ref.at[gidx_all.at[g]], rows_v.at[b],
                        sems[b]).wait()

                    @pl.when(jnp.logical_or(jnp.bool_(si > 0), g >= 2))
                    def _():
                        pltpu.make_async_copy(
                            ones_v, deg_sh.at[dst_all.at[g]], osems[b]).wait()

                    pltpu.async_copy(ones_v, deg_sh.at[dst_all.at[g]],
                                     osems[b], add=True)
                    pltpu.sync_copy(rows_v.at[b], agg_sh.at[dst_all.at[g]],
                                    add=True)

                @pl.when(g + 2 < sl)
                def _():
                    pltpu.async_copy(
                        table_ref.at[gidx_all.at[g + 2]], rows_v.at[b], sems[b])

            return carry

        lax.fori_loop(0, (sl + 1) // 2, body, 0)

    for b in range(2):
        pltpu.make_async_copy(ones_v, deg_sh.at[dst_all.at[b]], osems[b]).wait()
    plsc.subcore_barrier()

    def _out(off, cnt):
        off = pl.multiple_of(off, 8)
        pltpu.sync_copy(agg_sh.at[pl.ds(off, cnt)],
                        agg_out.at[cid, pl.ds(off, cnt)])
        pltpu.sync_copy(deg_sh.at[pl.ds(off, cnt)],
                        deg_out.at[cid, pl.ds(off, cnt)])

    _striped_rows(sid, _out)


def _sc_layer1(table1, gidx3, dst3, zagg, zdeg, ones):
    mesh = plsc.VectorSubcoreMesh(core_axis_name="c", subcore_axis_name="s")
    return pl.kernel(
        _sc_l1_body,
        mesh=mesh,
        out_type=[
            jax.ShapeDtypeStruct((NC, N, H), jnp.float32),
            jax.ShapeDtypeStruct((NC, N, 16), jnp.float32),
        ],
        scratch_types=[
            pltpu.VMEM((SC_HALF, KB1), jnp.int32),
            pltpu.VMEM((SC_HALF, KB1), jnp.int32),
            pltpu.VMEM((2, KB1, H), jnp.float32),
            pltpu.VMEM((KB1, 16), jnp.float32),
            pltpu.VMEM_SHARED((N, H), jnp.float32),
            pltpu.VMEM_SHARED((N, 16), jnp.float32),
            pltpu.SemaphoreType.DMA,
            pltpu.SemaphoreType.DMA,
            pltpu.SemaphoreType.DMA,
            pltpu.SemaphoreType.DMA,
        ],
        compiler_params=pltpu.CompilerParams(use_tc_tiling_on_sc=False),
    )(table1, gidx3, dst3, zagg, zdeg, ones)


def _l3_body(p1_ref, dp_ref, self1_ref, bases_ref, coeff_ref, wself_ref, b2_ref,
             table_ref, self2_ref):
    agg = p1_ref[0] + p1_ref[1]
    deg = dp_ref[0] + dp_ref[1]
    degc = jnp.max(deg, axis=1, keepdims=True)
    norm = 1.0 / jnp.maximum(degc, 1.0)
    h = jnp.maximum(agg * norm + self1_ref[...], 0.0)
    cols = []
    for r in range(R):
        w = coeff_ref[r, 0] * bases_ref[0]
        for b in range(1, B):
            w = w + coeff_ref[r, b] * bases_ref[b]
        cols.append(jnp.dot(h, w, preferred_element_type=jnp.float32))
    table_ref[...] = jnp.concatenate(cols, axis=1)
    self2_ref[...] = (
        jnp.dot(h, wself_ref[...], preferred_element_type=jnp.float32)
        + b2_ref[...]
    )


def _layer2_tables(p1, dp, self1, bases2, coeff2, wself2, b2r):
    bn = 1000
    nb = N // bn
    return pl.pallas_call(
        _l3_body,
        grid=(nb,),
        in_specs=[
            pl.BlockSpec((NC, bn, H), lambda i: (0, i, 0)),
            pl.BlockSpec((NC, bn, 16), lambda i: (0, i, 0)),
            pl.BlockSpec((bn, H), lambda i: (i, 0)),
            pl.BlockSpec((B, H, C), lambda i: (0, 0, 0)),
            pl.BlockSpec((R, B), lambda i: (0, 0)),
            pl.BlockSpec((H, C), lambda i: (0, 0)),
            pl.BlockSpec((1, C), lambda i: (0, 0)),
        ],
        out_specs=[
            pl.BlockSpec((bn, R * C), lambda i: (i, 0)),
            pl.BlockSpec((bn, C), lambda i: (i, 0)),
        ],
        out_shape=[
            jax.ShapeDtypeStruct((N, R * C), jnp.float32),
            jax.ShapeDtypeStruct((N, C), jnp.float32),
        ],
    )(p1, dp, self1, bases2, coeff2, wself2, b2r)


def _sc_l2_body(table_ref, gidx_ref, dst_ref, zagg_ref,
                agg_out,
                gidx_all, dst_all, rows_v, agg_sh,
                sem_a, sem_b, sem_c):
    cid = lax.axis_index("c")
    sid = lax.axis_index("s")
    wid = sid * NC + cid
    gsems = (sem_a, sem_b, sem_c)

    pltpu.sync_copy(gidx_ref.at[wid], gidx_all)
    pltpu.sync_copy(dst_ref.at[wid], dst_all)

    def _zero(off, cnt):
        off = pl.multiple_of(off, 8)
        pltpu.sync_copy(zagg_ref.at[pl.ds(off, cnt)], agg_sh.at[pl.ds(off, cnt)])

    _striped_rows(sid, _zero, last=RPT_LAST2)
    plsc.subcore_barrier()

    for b in range(2):
        pltpu.async_copy(table_ref.at[gidx_all.at[b]], rows_v.at[b], gsems[b])

    def body(j, carry):
        for b in range(3):
            g = j * 3 + b

            @pl.when(g < NITER2)
            def _():
                pltpu.make_async_copy(
                    table_ref.at[gidx_all.at[g]], rows_v.at[b], gsems[b]).wait()
                pltpu.async_copy(rows_v.at[b], agg_sh.at[dst_all.at[g]],
                                 gsems[b], add=True)

            bn = (b + 2) % 3

            @pl.when(g + 2 < NITER2)
            def _():
                @pl.when(g >= 1)
                def _():
                    pltpu.make_async_copy(
                        rows_v.at[bn], agg_sh.at[dst_all.at[g]],
                        gsems[bn]).wait()

                pltpu.async_copy(
                    table_ref.at[gidx_all.at[g + 2]], rows_v.at[bn], gsems[bn])

        return carry

    lax.fori_loop(0, (NITER2 + 2) // 3, body, 0)
    for b in range(3):
        g_last = NITER2 - 1
        pltpu.make_async_copy(
            rows_v.at[b], agg_sh.at[dst_all.at[g_last]], gsems[b]).wait()
    plsc.subcore_barrier()

    def _out(off, cnt):
        off = pl.multiple_of(off, 8)
        pltpu.sync_copy(agg_sh.at[pl.ds(off, cnt)],
                        agg_out.at[cid, pl.ds(off, cnt)])

    _striped_rows(sid, _out)


def _sc_layer2(table2, gidx3, dst3, zdeg):
    mesh = plsc.VectorSubcoreMesh(core_axis_name="c", subcore_axis_name="s")
    return pl.kernel(
        _sc_l2_body,
        mesh=mesh,
        out_type=jax.ShapeDtypeStruct((NC, N, C), jnp.float32),
        scratch_types=[
            pltpu.VMEM((NITER2, KB2), jnp.int32),
            pltpu.VMEM((NITER2, KB2), jnp.int32),
            pltpu.VMEM((3, KB2, C), jnp.float32),
            pltpu.VMEM_SHARED((N2, C), jnp.float32),
            pltpu.SemaphoreType.DMA,
            pltpu.SemaphoreType.DMA,
            pltpu.SemaphoreType.DMA,
        ],
        compiler_params=pltpu.CompilerParams(use_tc_tiling_on_sc=False),
    )(table2, gidx3, dst3, zdeg)


def _l5_body(p2_ref, dp_ref, self2_ref, o_ref):
    agg = p2_ref[0] + p2_ref[1]
    deg = dp_ref[0] + dp_ref[1]
    norm = 1.0 / jnp.maximum(deg, 1.0)
    logits = agg * norm + self2_ref[...]
    m = jnp.max(logits, axis=1, keepdims=True)
    e = jnp.exp(logits - m)
    o_ref[...] = e / jnp.sum(e, axis=1, keepdims=True)


def _final(p2, dp, self2):
    bn = 1000
    nb = N // bn
    return pl.pallas_call(
        _l5_body,
        grid=(nb,),
        in_specs=[
            pl.BlockSpec((NC, bn, C), lambda i: (0, i, 0)),
            pl.BlockSpec((NC, bn, 16), lambda i: (0, i, 0)),
            pl.BlockSpec((bn, C), lambda i: (i, 0)),
        ],
        out_specs=pl.BlockSpec((bn, C), lambda i: (i, 0)),
        out_shape=jax.ShapeDtypeStruct((N, C), jnp.float32),
    )(p2, dp, self2)


@jax.jit
def kernel(x, edge_index, edge_type, bases1, coeff1, wself1, b1,
           bases2, coeff2, wself2, b2):
    src2d = edge_index[0].reshape(E // 128, 128)
    et2d = edge_type.reshape(E // 128, 128)
    gidx1, gidx2 = _gidx(et2d, src2d)
    gidx1_3 = gidx1.reshape(NW, NITER1, KB1)
    dst3 = edge_index[1].reshape(NW, NITER1, KB1)
    gpad = jnp.zeros((EPAD - E,), jnp.int32)
    dpad = jnp.full((EPAD - E,), N, jnp.int32)
    gidx2_3 = jnp.concatenate([gidx2.reshape(E), gpad]).reshape(NW, NITER2, KB2)
    dst3b = jnp.concatenate([edge_index[1], dpad]).reshape(NW, NITER2, KB2)

    zagg = jnp.zeros((N, H), jnp.float32)
    zdeg = jnp.zeros((N, 16), jnp.float32)
    zdeg2 = jnp.zeros((N2, 16), jnp.float32)
    ones = jnp.ones((KB1, 16), jnp.float32)

    table1, self1 = _layer1_tables(x, bases1, coeff1, wself1, b1.reshape(1, H))
    p1, dp = _sc_layer1(table1.reshape(R * N, H), gidx1_3, dst3, zagg, zdeg, ones)
    table2p, self2 = _layer2_tables(
        p1, dp, self1, bases2, coeff2, wself2, b2.reshape(1, C))
    p2 = _sc_layer2(table2p.reshape(N * R, C), gidx2_3, dst3b, zdeg2)
    return _final(p2, dp, self2)


# speedup vs baseline: 1.0875x; 1.0677x over previous
import functools

import jax
import jax.numpy as jnp
from jax import lax
from jax.experimental import pallas as pl
from jax.experimental.pallas import tpu as pltpu
from jax.experimental.pallas import tpu_sc as plsc

N = 10000
E = 320000
D = 128
H = 128
C = 16
R = 8
B = 4

NC = 2
NS = 16
NW = NC * NS
KB1 = 80
NITER1 = E // (NW * KB1)
KB2 = 128
NITER2 = -(-E // (NW * KB2))
EPAD = NW * NITER2 * KB2
N2 = N + 8
RPT = 624
RPT_LAST = N - (NS - 1) * RPT
RPT_LAST2 = N2 - (NS - 1) * RPT
SC_STG = 56
SC_STAGES = ((0, SC_STG), (SC_STG, SC_STG), (2 * SC_STG, NITER1 - 2 * SC_STG))


def _striped_rows(sid, copy, last=RPT_LAST):
    @pl.when(sid < NS - 1)
    def _():
        copy(sid * RPT, RPT)

    @pl.when(sid == NS - 1)
    def _():
        copy((NS - 1) * RPT, last)


def _gidx_body(et_ref, src_ref, o1_ref, o2_ref):
    o1_ref[...] = et_ref[...] * N + src_ref[...]
    o2_ref[...] = src_ref[...] * R + et_ref[...]


def _gidx(et2d, src2d):
    nrows = et2d.shape[0]
    return pl.pallas_call(
        _gidx_body,
        grid=(1,),
        in_specs=[pl.BlockSpec((nrows, 128), lambda i: (0, 0))] * 2,
        out_specs=[pl.BlockSpec((nrows, 128), lambda i: (0, 0))] * 2,
        out_shape=[jax.ShapeDtypeStruct((nrows, 128), jnp.int32)] * 2,
    )(et2d, src2d)


def _l1_body(x_ref, bases_ref, coeff_ref, wself_ref, b1_ref, table_ref, self_ref):
    xb = x_ref[...]
    for r in range(R):
        w = coeff_ref[r, 0] * bases_ref[0]
        for b in range(1, B):
            w = w + coeff_ref[r, b] * bases_ref[b]
        table_ref[r] = jnp.dot(xb, w, preferred_element_type=jnp.float32)
    self_ref[...] = (
        jnp.dot(xb, wself_ref[...], preferred_element_type=jnp.float32)
        + b1_ref[...]
    )


def _layer1_tables(x, bases1, coeff1, wself1, b1r):
    bn = 1000
    nb = N // bn
    return pl.pallas_call(
        _l1_body,
        grid=(nb,),
        in_specs=[
            pl.BlockSpec((bn, D), lambda i: (i, 0)),
            pl.BlockSpec((B, D, H), lambda i: (0, 0, 0)),
            pl.BlockSpec((R, B), lambda i: (0, 0)),
            pl.BlockSpec((D, H), lambda i: (0, 0)),
            pl.BlockSpec((1, H), lambda i: (0, 0)),
        ],
        out_specs=[
            pl.BlockSpec((R, bn, H), lambda i: (0, i, 0)),
            pl.BlockSpec((bn, H), lambda i: (i, 0)),
        ],
        out_shape=[
            jax.ShapeDtypeStruct((R, N, H), jnp.float32),
            jax.ShapeDtypeStruct((N, H), jnp.float32),
        ],
    )(x, bases1, coeff1, wself1, b1r)


def _sc_l1_body(table_ref, gidx_ref, dst_ref, zagg_ref, zdeg_ref, ones_ref,
                agg_out, deg_out,
                gidx_all, dst_all, rows_v, ones_v, agg_sh, deg_sh,
                sem_a, sem_b, sem_c, sem_oa, sem_ob, sem_oc):
    cid = lax.axis_index("c")
    sid = lax.axis_index("s")
    wid = sid * NC + cid
    gsems = (sem_a, sem_b, sem_c)
    osems = (sem_oa, sem_ob, sem_oc)

    def _zero(off, cnt):
        off = pl.multiple_of(off, 8)
        pltpu.sync_copy(zagg_ref.at[pl.ds(off, cnt)], agg_sh.at[pl.ds(off, cnt)])
        pltpu.sync_copy(zdeg_ref.at[pl.ds(off, cnt)], deg_sh.at[pl.ds(off, cnt)])

    _striped_rows(sid, _zero)
    pltpu.sync_copy(ones_ref, ones_v)
    plsc.subcore_barrier()

    for si, (start, sl) in enumerate(SC_STAGES):
        pltpu.sync_copy(gidx_ref.at[wid, pl.ds(start, sl)],
                        gidx_all.at[pl.ds(0, sl)])
        pltpu.sync_copy(dst_ref.at[wid, pl.ds(start, sl)],
                        dst_all.at[pl.ds(0, sl)])
        for b in range(2):
            pltpu.async_copy(table_ref.at[gidx_all.at[b]], rows_v.at[b],
                             gsems[b])

        def body(j, carry):
            for b in range(3):
                g = j * 3 + b

                @pl.when(g < sl)
                def _():
                    pltpu.make_async_copy(
                        table_ref.at[gidx_all.at[g]], rows_v.at[b],
                        gsems[b]).wait()

                    @pl.when(jnp.logical_or(jnp.bool_(si > 0), g >= 3))
                    def _():
                        pltpu.make_async_copy(
                            ones_v, deg_sh.at[dst_all.at[g]], osems[b]).wait()

                    pltpu.async_copy(ones_v, deg_sh.at[dst_all.at[g]],
                                     osems[b], add=True)
                    pltpu.async_copy(rows_v.at[b], agg_sh.at[dst_all.at[g]],
                                     gsems[b], add=True)

                bn = (b + 2) % 3

                @pl.when(g + 2 < sl)
                def _():
                    @pl.when(g >= 1)
                    def _():
                        pltpu.make_async_copy(
                            rows_v.at[bn], agg_sh.at[dst_all.at[g]],
                            gsems[bn]).wait()

                    pltpu.async_copy(table_ref.at[gidx_all.at[g + 2]],
                                     rows_v.at[bn], gsems[bn])

            return carry

        lax.fori_loop(0, (sl + 2) // 3, body, 0)
        for b in range(3):
            pltpu.make_async_copy(
                rows_v.at[b], agg_sh.at[dst_all.at[0]], gsems[b]).wait()

    for b in range(3):
        pltpu.make_async_copy(ones_v, deg_sh.at[dst_all.at[b]], osems[b]).wait()
    plsc.subcore_barrier()

    def _out(off, cnt):
        off = pl.multiple_of(off, 8)
        pltpu.sync_copy(agg_sh.at[pl.ds(off, cnt)],
                        agg_out.at[cid, pl.ds(off, cnt)])
        pltpu.sync_copy(deg_sh.at[pl.ds(off, cnt)],
                        deg_out.at[cid, pl.ds(off, cnt)])

    _striped_rows(sid, _out)


def _sc_layer1(table1, gidx3, dst3, zagg, zdeg, ones):
    mesh = plsc.VectorSubcoreMesh(core_axis_name="c", subcore_axis_name="s")
    return pl.kernel(
        _sc_l1_body,
        mesh=mesh,
        out_type=[
            jax.ShapeDtypeStruct((NC, N, H), jnp.float32),
            jax.ShapeDtypeStruct((NC, N, 16), jnp.float32),
        ],
        scratch_types=[
            pltpu.VMEM((SC_STG, KB1), jnp.int32),
            pltpu.VMEM((SC_STG, KB1), jnp.int32),
            pltpu.VMEM((3, KB1, H), jnp.float32),
            pltpu.VMEM((KB1, 16), jnp.float32),
            pltpu.VMEM_SHARED((N, H), jnp.float32),
            pltpu.VMEM_SHARED((N, 16), jnp.float32),
            pltpu.SemaphoreType.DMA,
            pltpu.SemaphoreType.DMA,
            pltpu.SemaphoreType.DMA,
            pltpu.SemaphoreType.DMA,
            pltpu.SemaphoreType.DMA,
            pltpu.SemaphoreType.DMA,
        ],
        compiler_params=pltpu.CompilerParams(use_tc_tiling_on_sc=False),
    )(table1, gidx3, dst3, zagg, zdeg, ones)


def _l3_body(p1_ref, dp_ref, self1_ref, bases_ref, coeff_ref, wself_ref, b2_ref,
             table_ref, self2_ref):
    agg = p1_ref[0] + p1_ref[1]
    deg = dp_ref[0] + dp_ref[1]
    degc = jnp.max(deg, axis=1, keepdims=True)
    norm = 1.0 / jnp.maximum(degc, 1.0)
    h = jnp.maximum(agg * norm + self1_ref[...], 0.0)
    cols = []
    for r in range(R):
        w = coeff_ref[r, 0] * bases_ref[0]
        for b in range(1, B):
            w = w + coeff_ref[r, b] * bases_ref[b]
        cols.append(jnp.dot(h, w, preferred_element_type=jnp.float32))
    table_ref[...] = jnp.concatenate(cols, axis=1)
    self2_ref[...] = (
        jnp.dot(h, wself_ref[...], preferred_element_type=jnp.float32)
        + b2_ref[...]
    )


def _layer2_tables(p1, dp, self1, bases2, coeff2, wself2, b2r):
    bn = 1000
    nb = N // bn
    return pl.pallas_call(
        _l3_body,
        grid=(nb,),
        in_specs=[
            pl.BlockSpec((NC, bn, H), lambda i: (0, i, 0)),
            pl.BlockSpec((NC, bn, 16), lambda i: (0, i, 0)),
            pl.BlockSpec((bn, H), lambda i: (i, 0)),
            pl.BlockSpec((B, H, C), lambda i: (0, 0, 0)),
            pl.BlockSpec((R, B), lambda i: (0, 0)),
            pl.BlockSpec((H, C), lambda i: (0, 0)),
            pl.BlockSpec((1, C), lambda i: (0, 0)),
        ],
        out_specs=[
            pl.BlockSpec((bn, R * C), lambda i: (i, 0)),
            pl.BlockSpec((bn, C), lambda i: (i, 0)),
        ],
        out_shape=[
            jax.ShapeDtypeStruct((N, R * C), jnp.float32),
            jax.ShapeDtypeStruct((N, C), jnp.float32),
        ],
    )(p1, dp, self1, bases2, coeff2, wself2, b2r)


def _sc_l2_body(table_ref, gidx_ref, dst_ref, zagg_ref,
                agg_out,
                gidx_all, dst_all, rows_v, agg_sh,
                sem_a, sem_b, sem_c):
    cid = lax.axis_index("c")
    sid = lax.axis_index("s")
    wid = sid * NC + cid
    gsems = (sem_a, sem_b, sem_c)

    pltpu.sync_copy(gidx_ref.at[wid], gidx_all)
    pltpu.sync_copy(dst_ref.at[wid], dst_all)

    def _zero(off, cnt):
        off = pl.multiple_of(off, 8)
        pltpu.sync_copy(zagg_ref.at[pl.ds(off, cnt)], agg_sh.at[pl.ds(off, cnt)])

    _striped_rows(sid, _zero, last=RPT_LAST2)
    plsc.subcore_barrier()

    for b in range(2):
        pltpu.async_copy(table_ref.at[gidx_all.at[b]], rows_v.at[b], gsems[b])

    def body(j, carry):
        for b in range(3):
            g = j * 3 + b

            @pl.when(g < NITER2)
            def _():
                pltpu.make_async_copy(
                    table_ref.at[gidx_all.at[g]], rows_v.at[b], gsems[b]).wait()
                pltpu.async_copy(rows_v.at[b], agg_sh.at[dst_all.at[g]],
                                 gsems[b], add=True)

            bn = (b + 2) % 3

            @pl.when(g + 2 < NITER2)
            def _():
                @pl.when(g >= 1)
                def _():
                    pltpu.make_async_copy(
                        rows_v.at[bn], agg_sh.at[dst_all.at[g]],
                        gsems[bn]).wait()

                pltpu.async_copy(
                    table_ref.at[gidx_all.at[g + 2]], rows_v.at[bn], gsems[bn])

        return carry

    lax.fori_loop(0, (NITER2 + 2) // 3, body, 0)
    for b in range(3):
        g_last = NITER2 - 1
        pltpu.make_async_copy(
            rows_v.at[b], agg_sh.at[dst_all.at[g_last]], gsems[b]).wait()
    plsc.subcore_barrier()

    def _out(off, cnt):
        off = pl.multiple_of(off, 8)
        pltpu.sync_copy(agg_sh.at[pl.ds(off, cnt)],
                        agg_out.at[cid, pl.ds(off, cnt)])

    _striped_rows(sid, _out)


def _sc_layer2(table2, gidx3, dst3, zdeg):
    mesh = plsc.VectorSubcoreMesh(core_axis_name="c", subcore_axis_name="s")
    return pl.kernel(
        _sc_l2_body,
        mesh=mesh,
        out_type=jax.ShapeDtypeStruct((NC, N, C), jnp.float32),
        scratch_types=[
            pltpu.VMEM((NITER2, KB2), jnp.int32),
            pltpu.VMEM((NITER2, KB2), jnp.int32),
            pltpu.VMEM((3, KB2, C), jnp.float32),
            pltpu.VMEM_SHARED((N2, C), jnp.float32),
            pltpu.SemaphoreType.DMA,
            pltpu.SemaphoreType.DMA,
            pltpu.SemaphoreType.DMA,
        ],
        compiler_params=pltpu.CompilerParams(use_tc_tiling_on_sc=False),
    )(table2, gidx3, dst3, zdeg)


def _l5_body(p2_ref, dp_ref, self2_ref, o_ref):
    agg = p2_ref[0] + p2_ref[1]
    deg = dp_ref[0] + dp_ref[1]
    norm = 1.0 / jnp.maximum(deg, 1.0)
    logits = agg * norm + self2_ref[...]
    m = jnp.max(logits, axis=1, keepdims=True)
    e = jnp.exp(logits - m)
    o_ref[...] = e / jnp.sum(e, axis=1, keepdims=True)


def _final(p2, dp, self2):
    bn = 1000
    nb = N // bn
    return pl.pallas_call(
        _l5_body,
        grid=(nb,),
        in_specs=[
            pl.BlockSpec((NC, bn, C), lambda i: (0, i, 0)),
            pl.BlockSpec((NC, bn, 16), lambda i: (0, i, 0)),
            pl.BlockSpec((bn, C), lambda i: (i, 0)),
        ],
        out_specs=pl.BlockSpec((bn, C), lambda i: (i, 0)),
        out_shape=jax.ShapeDtypeStruct((N, C), jnp.float32),
    )(p2, dp, self2)


@jax.jit
def kernel(x, edge_index, edge_type, bases1, coeff1, wself1, b1,
           bases2, coeff2, wself2, b2):
    src2d = edge_index[0].reshape(E // 128, 128)
    et2d = edge_type.reshape(E // 128, 128)
    gidx1, gidx2 = _gidx(et2d, src2d)
    gidx1_3 = gidx1.reshape(NW, NITER1, KB1)
    dst3 = edge_index[1].reshape(NW, NITER1, KB1)
    gpad = jnp.zeros((EPAD - E,), jnp.int32)
    dpad = jnp.full((EPAD - E,), N, jnp.int32)
    gidx2_3 = jnp.concatenate([gidx2.reshape(E), gpad]).reshape(NW, NITER2, KB2)
    dst3b = jnp.concatenate([edge_index[1], dpad]).reshape(NW, NITER2, KB2)

    zagg = jnp.zeros((N, H), jnp.float32)
    zdeg = jnp.zeros((N, 16), jnp.float32)
    zdeg2 = jnp.zeros((N2, 16), jnp.float32)
    ones = jnp.ones((KB1, 16), jnp.float32)

    table1, self1 = _layer1_tables(x, bases1, coeff1, wself1, b1.reshape(1, H))
    p1, dp = _sc_layer1(table1.reshape(R * N, H), gidx1_3, dst3, zagg, zdeg, ones)
    table2p, self2 = _layer2_tables(
        p1, dp, self1, bases2, coeff2, wself2, b2.reshape(1, C))
    p2 = _sc_layer2(table2p.reshape(N * R, C), gidx2_3, dst3b, zdeg2)
    return _final(p2, dp, self2)
